# baseline (speedup 1.0000x reference)
import jax
import jax.numpy as jnp
from jax.experimental import pallas as pl
from jax.experimental.pallas import tpu as pltpu

_N = 4096
_BK = 1024
_NBLK = _N // _BK
_CH = 64
_LBD = 1.0


def _wtile(xsj, xsk, gj, gk):
    d = gj - gk
    sgn = jnp.minimum(jnp.maximum(d * 1e38, -1.0), 1.0)
    return jnp.tanh(xsj - xsk) - sgn


def _body(pred_row, gt_row, pred_col, gt_col, out_ref, xs_row, col_acc, acc_ref):
    i = pl.program_id(0)
    n = jnp.float32(_N)

    @pl.when(i == 0)
    def _init():
        x = pred_row[:, :]
        g = gt_row[:, :]
        sum_x = jnp.sum(x)
        sum_x2 = jnp.sum(x * x)
        pos = jax.lax.broadcasted_iota(jnp.int32, (1, _N), 1).astype(jnp.float32)
        s1 = jnp.sum(x * (2.0 * pos - (n - 1.0)))
        ss = n * sum_x2 - sum_x * sum_x
        var = (ss - s1 * s1 / (n * n)) / (n * n - 1.0)
        s2 = 3.4 / jnp.sqrt(var)
        acc_ref[0] = s2
        acc_ref[1] = jnp.sum(jnp.abs(x - g)) / n
        acc_ref[2] = 0.0
        xs_row[:, :] = x * s2
        col_acc[:, :] = jnp.zeros((1, _N), jnp.float32)

    s2 = acc_ref[0]
    xsk = pred_col[:, :] * s2
    gk = gt_col[:, :]

    def _tile_sums(xsj, gj, with_col):
        rows = []
        colp = jnp.zeros((1, _BK), jnp.float32)
        for c in range(0, _BK, _CH):
            w_c = _wtile(xsj, xsk[c:c + _CH, :], gj, gk[c:c + _CH, :])
            rows.append(jnp.sum(w_c, axis=1, keepdims=True))
            if with_col:
                colp = colp + jnp.sum(w_c, axis=0, keepdims=True)
        return jnp.concatenate(rows, axis=0), colp

    off_d = pl.multiple_of(i * _BK, _BK)
    xsj_d = xs_row[0:1, pl.ds(off_d, _BK)]
    gj_d = gt_row[0:1, pl.ds(off_d, _BK)]
    wrow0, _ = _tile_sums(xsj_d, gj_d, with_col=False)

    def _loop(nb, wrow):
        off = pl.multiple_of(nb * _BK, _BK)
        xsj = xs_row[0:1, pl.ds(off, _BK)]
        gj = gt_row[0:1, pl.ds(off, _BK)]
        rowp, colp = _tile_sums(xsj, gj, with_col=True)
        col_acc[0:1, pl.ds(off, _BK)] -= colp
        return wrow + rowp

    wrow = jax.lax.fori_loop(i + 1, _NBLK, _loop, wrow0)
    wk = jnp.swapaxes(wrow, 0, 1) + col_acc[0:1, pl.ds(off_d, _BK)]
    acc_ref[2] += jnp.sum(wk * wk)

    @pl.when(i == _NBLK - 1)
    def _fin():
        out_ref[0] = acc_ref[2] / (4.0 * n * n * n) + _LBD * acc_ref[1]


def kernel(mem_pred, mem_gt):
    pred_row = mem_pred.reshape(1, _N)
    gt_row = mem_gt.reshape(1, _N)
    pred_col = mem_pred.reshape(_N, 1)
    gt_col = mem_gt.reshape(_N, 1)

    out = pl.pallas_call(
        _body,
        grid=(_NBLK,),
        in_specs=[
            pl.BlockSpec((1, _N), lambda i: (0, 0)),
            pl.BlockSpec((1, _N), lambda i: (0, 0)),
            pl.BlockSpec((_BK, 1), lambda i: (i, 0)),
            pl.BlockSpec((_BK, 1), lambda i: (i, 0)),
        ],
        out_specs=pl.BlockSpec(memory_space=pltpu.SMEM),
        out_shape=jax.ShapeDtypeStruct((1,), jnp.float32),
        scratch_shapes=[
            pltpu.VMEM((1, _N), jnp.float32),
            pltpu.VMEM((1, _N), jnp.float32),
            pltpu.SMEM((3,), jnp.float32),
        ],
    )(pred_row, gt_row, pred_col, gt_col)
    return out[0]


# speedup vs baseline: 1.3633x; 1.3633x over previous
import jax
import jax.numpy as jnp
from jax.experimental import pallas as pl
from jax.experimental.pallas import tpu as pltpu

_N = 4096
_BK = 1024
_NBLK = _N // _BK
_LBD = 1.0


def _wtile(xsj, xsk, gj, gk):
    d = gj - gk
    sgn = jnp.minimum(jnp.maximum(d * 1e38, -1.0), 1.0)
    return jnp.tanh(xsj - xsk) - sgn


def _body(pred_row, gt_row, pred_col, gt_col, out_ref, xs_row, col_acc, acc_ref):
    i = pl.program_id(0)
    n = jnp.float32(_N)

    @pl.when(i == 0)
    def _init():
        x = pred_row[:, :]
        g = gt_row[:, :]
        sum_x = jnp.sum(x)
        sum_x2 = jnp.sum(x * x)
        pos = jax.lax.broadcasted_iota(jnp.int32, (1, _N), 1).astype(jnp.float32)
        s1 = jnp.sum(x * (2.0 * pos - (n - 1.0)))
        ss = n * sum_x2 - sum_x * sum_x
        var = (ss - s1 * s1 / (n * n)) / (n * n - 1.0)
        s2 = 3.4 / jnp.sqrt(var)
        acc_ref[0] = s2
        acc_ref[1] = jnp.sum(jnp.abs(x - g)) / n
        acc_ref[2] = 0.0
        xs_row[:, :] = x * s2
        col_acc[:, :] = jnp.zeros((1, _N), jnp.float32)

    s2 = acc_ref[0]
    xsk = pred_col[:, :] * s2
    gk = gt_col[:, :]

    off_d = pl.multiple_of(i * _BK, _BK)
    xsj_d = xs_row[0:1, pl.ds(off_d, _BK)]
    gj_d = gt_row[0:1, pl.ds(off_d, _BK)]
    wrow0 = jnp.sum(_wtile(xsj_d, xsk, gj_d, gk), axis=1, keepdims=True)

    def _loop(nb, wrow):
        off = pl.multiple_of(nb * _BK, _BK)
        xsj = xs_row[0:1, pl.ds(off, _BK)]
        gj = gt_row[0:1, pl.ds(off, _BK)]
        w = _wtile(xsj, xsk, gj, gk)
        col_acc[0:1, pl.ds(off, _BK)] -= jnp.sum(w, axis=0, keepdims=True)
        return wrow + jnp.sum(w, axis=1, keepdims=True)

    wrow = jax.lax.fori_loop(i + 1, _NBLK, _loop, wrow0)
    wk = jnp.swapaxes(wrow, 0, 1) + col_acc[0:1, pl.ds(off_d, _BK)]
    acc_ref[2] += jnp.sum(wk * wk)

    @pl.when(i == _NBLK - 1)
    def _fin():
        out_ref[0] = acc_ref[2] / (4.0 * n * n * n) + _LBD * acc_ref[1]


def kernel(mem_pred, mem_gt):
    pred_row = mem_pred.reshape(1, _N)
    gt_row = mem_gt.reshape(1, _N)
    pred_col = mem_pred.reshape(_N, 1)
    gt_col = mem_gt.reshape(_N, 1)

    out = pl.pallas_call(
        _body,
        grid=(_NBLK,),
        in_specs=[
            pl.BlockSpec((1, _N), lambda i: (0, 0)),
            pl.BlockSpec((1, _N), lambda i: (0, 0)),
            pl.BlockSpec((_BK, 1), lambda i: (i, 0)),
            pl.BlockSpec((_BK, 1), lambda i: (i, 0)),
        ],
        out_specs=pl.BlockSpec(memory_space=pltpu.SMEM),
        out_shape=jax.ShapeDtypeStruct((1,), jnp.float32),
        scratch_shapes=[
            pltpu.VMEM((1, _N), jnp.float32),
            pltpu.VMEM((1, _N), jnp.float32),
            pltpu.SMEM((3,), jnp.float32),
        ],
    )(pred_row, gt_row, pred_col, gt_col)
    return out[0]
